# baseline (speedup 1.0000x reference)
import functools

import jax
import jax.numpy as jnp
from jax import lax
from jax.experimental import pallas as pl
from jax.experimental.pallas import tpu as pltpu
from jax.experimental.pallas import tpu_sc as plsc

ALPHA = 0.1
NC = 2
NS = 16
NTEC = NC * NS
LANES = 16
CHUNK = 32
PREC = lax.Precision.HIGHEST


def _rsqrt16(d):
    i = plsc.bitcast(d, jnp.int32)
    i = jnp.int32(0x5F3759DF) - lax.shift_right_logical(i, 1)
    y = plsc.bitcast(i, jnp.float32)
    for _ in range(3):
        y = y * (1.5 - 0.5 * d * y * y)
    return y


def _deg_body(np_, ec, rows_hbm, ws_hbm, dpart_hbm, hist, rbuf, wbuf, tbuf,
              shared):
    cid = lax.axis_index("c")
    sid = lax.axis_index("s")
    wid = cid * NS + sid
    zz = jnp.zeros((LANES,), jnp.float32)

    @pl.loop(0, np_ // LANES)
    def _(i):
        hist[pl.ds(i * LANES, LANES)] = zz

    off = wid * ec
    pltpu.sync_copy(rows_hbm.at[pl.ds(off, ec)], rbuf)
    pltpu.sync_copy(ws_hbm.at[pl.ds(off, ec)], wbuf)
    lane = lax.iota(jnp.int32, LANES)

    @pl.loop(0, ec // LANES)
    def _(g):
        sl = pl.ds(g * LANES, LANES)
        idxv = rbuf[sl]
        wv = wbuf[sl]
        for l in range(LANES):
            plsc.addupdate_scatter(hist, [idxv], wv, mask=lane == l)

    pltpu.sync_copy(hist, shared.at[sid])
    plsc.subcore_barrier()

    slc = np_ // NS
    slo = sid * slc
    pltpu.sync_copy(shared.at[:, pl.ds(slo, slc)], tbuf)

    @pl.loop(0, slc // LANES)
    def _(i):
        sl = pl.ds(i * LANES, LANES)
        v = tbuf[0, sl]
        for t in range(1, NS):
            v = v + tbuf[t, sl]
        hist[sl] = v

    pltpu.sync_copy(hist.at[pl.ds(0, slc)], dpart_hbm.at[cid, pl.ds(slo, slc)])


def _norm_body(np_, ec, rows_hbm, cols_hbm, ws_hbm, dpart_hbm, norm_hbm,
               dtmp, dbuf, rbuf, cbuf, wbuf, obuf):
    cid = lax.axis_index("c")
    sid = lax.axis_index("s")
    wid = cid * NS + sid
    pltpu.sync_copy(dpart_hbm, dtmp)

    @pl.loop(0, np_ // LANES)
    def _(i):
        sl = pl.ds(i * LANES, LANES)
        dbuf[sl] = dtmp[0, sl] + dtmp[1, sl]

    off = wid * ec
    pltpu.sync_copy(rows_hbm.at[pl.ds(off, ec)], rbuf)
    pltpu.sync_copy(cols_hbm.at[pl.ds(off, ec)], cbuf)
    pltpu.sync_copy(ws_hbm.at[pl.ds(off, ec)], wbuf)

    @pl.loop(0, ec // LANES)
    def _(g):
        sl = pl.ds(g * LANES, LANES)
        dr = _rsqrt16(plsc.load_gather(dbuf, [rbuf[sl]]))
        dc = _rsqrt16(plsc.load_gather(dbuf, [cbuf[sl]]))
        obuf[sl] = dr * wbuf[sl] * dc

    pltpu.sync_copy(obuf, norm_hbm.at[pl.ds(off, ec)])


NBUF = 8
LA = 4
NSPLIT = 2


def _spmm_body(npt2, nsub, prev_hbm, rows_hbm, cols_hbm, nrm_hbm, st_hbm,
               agg_hbm, acc, gbuf, ibuf, dstb, nrb, sbuf, esem, gsem):
    cid = lax.axis_index("c")
    sid = lax.axis_index("s")
    wid = cid * NS + sid
    pltpu.sync_copy(st_hbm, sbuf)
    zz = jnp.zeros((LANES,), jnp.float32)
    lane = lax.iota(jnp.int32, LANES)

    @pl.loop(0, NSPLIT)
    def _(sp):
        vidx = wid * NSPLIT + sp
        sv = sbuf[pl.ds(vidx, LANES)]
        s_t = sv[0]
        e_t = sv[1]
        base = vidx * npt2

        @plsc.parallel_loop(0, npt2)
        def _(i):
            for k in range(nsub):
                acc[i, pl.ds(k * LANES, LANES)] = zz

        s8 = (s_t // 8) * 8
        nch = lax.div(e_t - s8 + (CHUNK - 1), CHUNK)

        def fetch(c, b):
            off = s8 + c * CHUNK
            pltpu.async_copy(rows_hbm.at[pl.ds(off, CHUNK)], ibuf.at[b],
                             esem.at[b])
            pltpu.async_copy(cols_hbm.at[pl.ds(off, CHUNK)], dstb.at[b],
                             esem.at[b])
            pltpu.async_copy(nrm_hbm.at[pl.ds(off, CHUNK)], nrb.at[b],
                             esem.at[b])

        def wait_e(b):
            pltpu.make_async_copy(rows_hbm.at[pl.ds(0, CHUNK)], ibuf.at[b],
                                  esem.at[b]).wait()
            pltpu.make_async_copy(cols_hbm.at[pl.ds(0, CHUNK)], dstb.at[b],
                                  esem.at[b]).wait()
            pltpu.make_async_copy(nrm_hbm.at[pl.ds(0, CHUNK)], nrb.at[b],
                                  esem.at[b]).wait()

        def gather(b):
            pltpu.async_copy(prev_hbm.at[ibuf.at[b]], gbuf.at[b], gsem.at[b])

        def wait_g(b):
            pltpu.make_async_copy(prev_hbm.at[ibuf.at[b]], gbuf.at[b],
                                  gsem.at[b]).wait()

        for k0 in range(NBUF):

            @pl.when(k0 < nch)
            def _():
                fetch(k0, k0)

        for k0 in range(LA):

            @pl.when(k0 < nch)
            def _():
                wait_e(k0)
                gather(k0)

        @pl.loop(0, lax.div(nch + NBUF - 1, NBUF))
        def _(o):
            for u in range(NBUF):
                c = o * NBUF + u

                @pl.when(c < nch)
                def _():

                    @pl.when(c + LA < nch)
                    def _():
                        wait_e((u + LA) % NBUF)
                        gather((u + LA) % NBUF)

                    wait_g(u)
                    cbase = s8 + c * CHUNK

                    @pl.loop(0, CHUNK // LANES)
                    def _(g):
                        gsl = pl.ds(g * LANES, LANES)
                        pos = lane + (cbase + g * LANES)
                        okv = (pos >= s_t) & (pos < e_t)
                        nrv = jnp.where(okv, nrb[u, gsl], 0.0)
                        relv = jnp.where(okv, dstb[u, gsl] - base, 0)
                        for l in range(LANES):
                            rel = relv[l]
                            nv = lax.broadcast(nrv[l], (LANES,))
                            j = g * LANES + l

                            @plsc.parallel_loop(0, nsub * LANES, step=LANES,
                                                unroll=nsub)
                            def _(co):
                                sl = pl.ds(co, LANES)
                                plsc.addupdate(acc.at[rel, sl],
                                               nv * gbuf[u, j, sl])

                    @pl.when(c + NBUF < nch)
                    def _():
                        fetch(c + NBUF, u)

        pltpu.sync_copy(acc, agg_hbm.at[pl.ds(base, npt2)])


def _mlp_body(nl, x_ref, w0_ref, b0_ref, w2_ref, bc_ref, h_ref, z_ref):
    h = jnp.dot(x_ref[...], w0_ref[...], preferred_element_type=jnp.float32,
                precision=PREC)
    h = jnp.maximum(h + b0_ref[...], 0.0)
    h_ref[...] = h
    ha = h * ALPHA
    for i in range(nl):
        z = jnp.dot(ha, w2_ref[i], preferred_element_type=jnp.float32,
                    precision=PREC)
        z_ref[i] = z + bc_ref[i]


def _layer_body(agg_ref, w1_ref, z_ref, prev_ref, out_ref):
    a = agg_ref[...] * (1.0 - ALPHA)
    o = jnp.dot(a, w1_ref[...], preferred_element_type=jnp.float32,
                precision=PREC) + z_ref[...]
    out_ref[...] = jnp.maximum(o, 0.0) + prev_ref[...]


def _final_body(agg_ref, w1_ref, z_ref, prev_ref, wo_ref, bo_ref, out_ref):
    a = agg_ref[...] * (1.0 - ALPHA)
    o = jnp.dot(a, w1_ref[...], preferred_element_type=jnp.float32,
                precision=PREC) + z_ref[...]
    p = jnp.maximum(o, 0.0) + prev_ref[...]
    lg = jnp.dot(p, wo_ref[...], preferred_element_type=jnp.float32,
                 precision=PREC) + bo_ref[...]
    m = jnp.max(lg, axis=1, keepdims=True)
    e = lg - m
    out_ref[...] = e - jnp.log(jnp.sum(jnp.exp(e), axis=1, keepdims=True))


def kernel(x, edge_index, edge_attr, W0, b0, W1, W2, bconv, Wout, bout):
    n, d_in = x.shape
    d_hid = W0.shape[1]
    d_out = Wout.shape[1]
    n_layers = W1.shape[0]
    nsub = d_hid // LANES
    nvw = NTEC * NSPLIT
    npt2 = -(-((n + nvw - 1) // nvw) // 8) * 8
    np_ = nvw * npt2

    row = edge_index[0].astype(jnp.int32)
    col = edge_index[1].astype(jnp.int32)
    loopidx = jnp.arange(n, dtype=jnp.int32)
    row2 = jnp.concatenate([row, loopidx])
    col2 = jnp.concatenate([col, loopidx])
    wall = jnp.concatenate(
        [edge_attr.astype(jnp.float32), jnp.ones((n,), jnp.float32)])
    perm = jnp.argsort(col2)
    rows_s = row2[perm]
    cols_s = col2[perm]
    ws_s = wall[perm]
    e2 = rows_s.shape[0]
    gran = NTEC * LANES
    e2p = -(-(e2 + 2 * CHUNK) // gran) * gran
    padn = e2p - e2
    rows_p = jnp.concatenate([rows_s, jnp.zeros((padn,), jnp.int32)])
    cols_p = jnp.concatenate([cols_s, jnp.zeros((padn,), jnp.int32)])
    ws_p = jnp.concatenate([ws_s, jnp.zeros((padn,), jnp.float32)])
    ec = e2p // NTEC

    bounds = jnp.arange(nvw + 1, dtype=jnp.int32) * npt2
    starts = jnp.searchsorted(cols_s, bounds).astype(jnp.int32)
    starts_p = jnp.concatenate([starts, jnp.zeros((15,), jnp.int32)])
    xp = jnp.pad(x, ((0, np_ - n), (0, 0)))

    mesh = plsc.VectorSubcoreMesh(core_axis_name="c", subcore_axis_name="s",
                                  num_cores=NC, num_subcores=NS)

    scp = pltpu.CompilerParams(needs_layout_passes=False)
    deg_k = pl.kernel(
        functools.partial(_deg_body, np_, ec),
        out_type=jax.ShapeDtypeStruct((NC, np_), jnp.float32),
        mesh=mesh,
        compiler_params=scp,
        scratch_types=[
            pltpu.VMEM((np_,), jnp.float32),
            pltpu.VMEM((ec,), jnp.int32),
            pltpu.VMEM((ec,), jnp.float32),
            pltpu.VMEM((NS, np_ // NS), jnp.float32),
            pltpu.VMEM_SHARED((NS, np_), jnp.float32),
        ],
    )
    dpart = deg_k(rows_p, ws_p)

    norm_k = pl.kernel(
        functools.partial(_norm_body, np_, ec),
        out_type=jax.ShapeDtypeStruct((e2p,), jnp.float32),
        mesh=mesh,
        compiler_params=scp,
        scratch_types=[
            pltpu.VMEM((NC, np_), jnp.float32),
            pltpu.VMEM((np_,), jnp.float32),
            pltpu.VMEM((ec,), jnp.int32),
            pltpu.VMEM((ec,), jnp.int32),
            pltpu.VMEM((ec,), jnp.float32),
            pltpu.VMEM((ec,), jnp.float32),
        ],
    )
    normv = norm_k(rows_p, cols_p, ws_p, dpart)

    spmm_k = pl.kernel(
        functools.partial(_spmm_body, npt2, nsub),
        out_type=jax.ShapeDtypeStruct((np_, d_hid), jnp.float32),
        mesh=mesh,
        compiler_params=scp,
        scratch_types=[
            pltpu.VMEM((npt2, d_hid), jnp.float32),
            pltpu.VMEM((NBUF, CHUNK, d_hid), jnp.float32),
            pltpu.VMEM((NBUF, CHUNK), jnp.int32),
            pltpu.VMEM((NBUF, CHUNK), jnp.int32),
            pltpu.VMEM((NBUF, CHUNK), jnp.float32),
            pltpu.VMEM((nvw + 16,), jnp.int32),
            pltpu.SemaphoreType.DMA((NBUF,)),
            pltpu.SemaphoreType.DMA((NBUF,)),
        ],
    )

    bm = 512 if np_ % 512 == 0 else npt
    grid = (np_ // bm,)
    h0, zs = pl.pallas_call(
        functools.partial(_mlp_body, n_layers),
        grid=grid,
        in_specs=[
            pl.BlockSpec((bm, d_in), lambda i: (i, 0)),
            pl.BlockSpec((d_in, d_hid), lambda i: (0, 0)),
            pl.BlockSpec((1, d_hid), lambda i: (0, 0)),
            pl.BlockSpec((n_layers, d_hid, d_hid), lambda i: (0, 0, 0)),
            pl.BlockSpec((n_layers, d_hid), lambda i: (0, 0)),
        ],
        out_specs=[
            pl.BlockSpec((bm, d_hid), lambda i: (i, 0)),
            pl.BlockSpec((n_layers, bm, d_hid), lambda i: (0, i, 0)),
        ],
        out_shape=[
            jax.ShapeDtypeStruct((np_, d_hid), jnp.float32),
            jax.ShapeDtypeStruct((n_layers, np_, d_hid), jnp.float32),
        ],
    )(xp, W0, b0.reshape(1, d_hid), W2, bconv)

    layer_k = pl.pallas_call(
        _layer_body,
        grid=grid,
        in_specs=[
            pl.BlockSpec((bm, d_hid), lambda i: (i, 0)),
            pl.BlockSpec((d_hid, d_hid), lambda i: (0, 0)),
            pl.BlockSpec((bm, d_hid), lambda i: (i, 0)),
            pl.BlockSpec((bm, d_hid), lambda i: (i, 0)),
        ],
        out_specs=pl.BlockSpec((bm, d_hid), lambda i: (i, 0)),
        out_shape=jax.ShapeDtypeStruct((np_, d_hid), jnp.float32),
    )

    final_k = pl.pallas_call(
        _final_body,
        grid=grid,
        in_specs=[
            pl.BlockSpec((bm, d_hid), lambda i: (i, 0)),
            pl.BlockSpec((d_hid, d_hid), lambda i: (0, 0)),
            pl.BlockSpec((bm, d_hid), lambda i: (i, 0)),
            pl.BlockSpec((bm, d_hid), lambda i: (i, 0)),
            pl.BlockSpec((d_hid, d_out), lambda i: (0, 0)),
            pl.BlockSpec((1, d_out), lambda i: (0, 0)),
        ],
        out_specs=pl.BlockSpec((bm, d_out), lambda i: (i, 0)),
        out_shape=jax.ShapeDtypeStruct((np_, d_out), jnp.float32),
    )

    prev = h0
    for i in range(n_layers):
        agg = spmm_k(prev, rows_p, cols_p, normv, starts_p)
        if i < n_layers - 1:
            prev = layer_k(agg, W1[i], zs[i], prev)
        else:
            out = final_k(agg, W1[i], zs[i], prev, Wout,
                          bout.reshape(1, d_out))
    return out[:n]


# speedup vs baseline: 1.3556x; 1.3556x over previous
import functools

import jax
import jax.numpy as jnp
from jax import lax
from jax.experimental import pallas as pl
from jax.experimental.pallas import tpu as pltpu
from jax.experimental.pallas import tpu_sc as plsc

ALPHA = 0.1
NC = 2
NS = 16
NTEC = NC * NS
LANES = 16
CHUNK = 128
PREC = lax.Precision.HIGHEST


def _rsqrt16(d):
    i = plsc.bitcast(d, jnp.int32)
    i = jnp.int32(0x5F3759DF) - lax.shift_right_logical(i, 1)
    y = plsc.bitcast(i, jnp.float32)
    for _ in range(3):
        y = y * (1.5 - 0.5 * d * y * y)
    return y


def _deg_body(np_, ec, rows_hbm, ws_hbm, dpart_hbm, hist, rbuf, wbuf, tbuf,
              shared):
    cid = lax.axis_index("c")
    sid = lax.axis_index("s")
    wid = cid * NS + sid
    zz = jnp.zeros((LANES,), jnp.float32)

    @pl.loop(0, np_ // LANES)
    def _(i):
        hist[pl.ds(i * LANES, LANES)] = zz

    off = wid * ec
    pltpu.sync_copy(rows_hbm.at[pl.ds(off, ec)], rbuf)
    pltpu.sync_copy(ws_hbm.at[pl.ds(off, ec)], wbuf)
    lane = lax.iota(jnp.int32, LANES)

    @pl.loop(0, ec // LANES)
    def _(g):
        sl = pl.ds(g * LANES, LANES)
        idxv = rbuf[sl]
        wv = wbuf[sl]
        for l in range(LANES):
            plsc.addupdate_scatter(hist, [idxv], wv, mask=lane == l)

    pltpu.sync_copy(hist, shared.at[sid])
    plsc.subcore_barrier()

    slc = np_ // NS
    slo = sid * slc
    pltpu.sync_copy(shared.at[:, pl.ds(slo, slc)], tbuf)

    @pl.loop(0, slc // LANES)
    def _(i):
        sl = pl.ds(i * LANES, LANES)
        v = tbuf[0, sl]
        for t in range(1, NS):
            v = v + tbuf[t, sl]
        hist[sl] = v

    pltpu.sync_copy(hist.at[pl.ds(0, slc)], dpart_hbm.at[cid, pl.ds(slo, slc)])


def _norm_body(np_, ec, rows_hbm, cols_hbm, ws_hbm, dpart_hbm, norm_hbm,
               dtmp, dbuf, rbuf, cbuf, wbuf, obuf):
    cid = lax.axis_index("c")
    sid = lax.axis_index("s")
    wid = cid * NS + sid
    pltpu.sync_copy(dpart_hbm, dtmp)

    @pl.loop(0, np_ // LANES)
    def _(i):
        sl = pl.ds(i * LANES, LANES)
        dbuf[sl] = dtmp[0, sl] + dtmp[1, sl]

    off = wid * ec
    pltpu.sync_copy(rows_hbm.at[pl.ds(off, ec)], rbuf)
    pltpu.sync_copy(cols_hbm.at[pl.ds(off, ec)], cbuf)
    pltpu.sync_copy(ws_hbm.at[pl.ds(off, ec)], wbuf)

    @pl.loop(0, ec // LANES)
    def _(g):
        sl = pl.ds(g * LANES, LANES)
        dr = _rsqrt16(plsc.load_gather(dbuf, [rbuf[sl]]))
        dc = _rsqrt16(plsc.load_gather(dbuf, [cbuf[sl]]))
        obuf[sl] = dr * wbuf[sl] * dc

    pltpu.sync_copy(obuf, norm_hbm.at[pl.ds(off, ec)])


NBUF = 2
LA = 1
NSPLIT = 2


def _spmm_body(npt2, nsub, prev_hbm, rows_hbm, cols_hbm, nrm_hbm, st_hbm,
               agg_hbm, acc, gbuf, ibuf, dstb, nrb, sbuf, esem, gsem):
    cid = lax.axis_index("c")
    sid = lax.axis_index("s")
    wid = cid * NS + sid
    pltpu.sync_copy(st_hbm, sbuf)
    zz = jnp.zeros((LANES,), jnp.float32)
    lane = lax.iota(jnp.int32, LANES)

    @pl.loop(0, NSPLIT)
    def _(sp):
        vidx = wid * NSPLIT + sp
        sv = sbuf[pl.ds(vidx, LANES)]
        s_t = sv[0]
        e_t = sv[1]
        base = vidx * npt2

        @plsc.parallel_loop(0, npt2)
        def _(i):
            for k in range(nsub):
                acc[i, pl.ds(k * LANES, LANES)] = zz

        s8 = (s_t // 8) * 8
        nch = lax.div(e_t - s8 + (CHUNK - 1), CHUNK)

        def fetch(c, b):
            off = s8 + c * CHUNK
            pltpu.async_copy(rows_hbm.at[pl.ds(off, CHUNK)], ibuf.at[b],
                             esem.at[b])
            pltpu.async_copy(cols_hbm.at[pl.ds(off, CHUNK)], dstb.at[b],
                             esem.at[b])
            pltpu.async_copy(nrm_hbm.at[pl.ds(off, CHUNK)], nrb.at[b],
                             esem.at[b])

        def wait_e(b):
            pltpu.make_async_copy(rows_hbm.at[pl.ds(0, CHUNK)], ibuf.at[b],
                                  esem.at[b]).wait()
            pltpu.make_async_copy(cols_hbm.at[pl.ds(0, CHUNK)], dstb.at[b],
                                  esem.at[b]).wait()
            pltpu.make_async_copy(nrm_hbm.at[pl.ds(0, CHUNK)], nrb.at[b],
                                  esem.at[b]).wait()

        def gather(b):
            pltpu.async_copy(prev_hbm.at[ibuf.at[b]], gbuf.at[b], gsem.at[b])

        def wait_g(b):
            pltpu.make_async_copy(prev_hbm.at[ibuf.at[b]], gbuf.at[b],
                                  gsem.at[b]).wait()

        for k0 in range(NBUF):

            @pl.when(k0 < nch)
            def _():
                fetch(k0, k0)

        for k0 in range(LA):

            @pl.when(k0 < nch)
            def _():
                wait_e(k0)
                gather(k0)

        @pl.loop(0, lax.div(nch + NBUF - 1, NBUF))
        def _(o):
            for u in range(NBUF):
                c = o * NBUF + u

                @pl.when(c < nch)
                def _():

                    @pl.when(c + LA < nch)
                    def _():
                        wait_e((u + LA) % NBUF)
                        gather((u + LA) % NBUF)

                    wait_g(u)
                    cbase = s8 + c * CHUNK

                    @pl.loop(0, CHUNK // LANES)
                    def _(g):
                        gsl = pl.ds(g * LANES, LANES)
                        pos = lane + (cbase + g * LANES)
                        okv = (pos >= s_t) & (pos < e_t)
                        nrv = jnp.where(okv, nrb[u, gsl], 0.0)
                        relv = jnp.where(okv, dstb[u, gsl] - base, 0)
                        for l in range(LANES):
                            rel = relv[l]
                            nv = lax.broadcast(nrv[l], (LANES,))
                            j = g * LANES + l

                            @plsc.parallel_loop(0, nsub * LANES, step=LANES,
                                                unroll=nsub)
                            def _(co):
                                sl = pl.ds(co, LANES)
                                plsc.addupdate(acc.at[rel, sl],
                                               nv * gbuf[u, j, sl])

                    @pl.when(c + NBUF < nch)
                    def _():
                        fetch(c + NBUF, u)

        pltpu.sync_copy(acc, agg_hbm.at[pl.ds(base, npt2)])


def _mlp_body(nl, x_ref, w0_ref, b0_ref, w2_ref, bc_ref, h_ref, z_ref):
    h = jnp.dot(x_ref[...], w0_ref[...], preferred_element_type=jnp.float32,
                precision=PREC)
    h = jnp.maximum(h + b0_ref[...], 0.0)
    h_ref[...] = h
    ha = h * ALPHA
    for i in range(nl):
        z = jnp.dot(ha, w2_ref[i], preferred_element_type=jnp.float32,
                    precision=PREC)
        z_ref[i] = z + bc_ref[i]


def _layer_body(agg_ref, w1_ref, z_ref, prev_ref, out_ref):
    a = agg_ref[...] * (1.0 - ALPHA)
    o = jnp.dot(a, w1_ref[...], preferred_element_type=jnp.float32,
                precision=PREC) + z_ref[...]
    out_ref[...] = jnp.maximum(o, 0.0) + prev_ref[...]


def _final_body(agg_ref, w1_ref, z_ref, prev_ref, wo_ref, bo_ref, out_ref):
    a = agg_ref[...] * (1.0 - ALPHA)
    o = jnp.dot(a, w1_ref[...], preferred_element_type=jnp.float32,
                precision=PREC) + z_ref[...]
    p = jnp.maximum(o, 0.0) + prev_ref[...]
    lg = jnp.dot(p, wo_ref[...], preferred_element_type=jnp.float32,
                 precision=PREC) + bo_ref[...]
    m = jnp.max(lg, axis=1, keepdims=True)
    e = lg - m
    out_ref[...] = e - jnp.log(jnp.sum(jnp.exp(e), axis=1, keepdims=True))


def kernel(x, edge_index, edge_attr, W0, b0, W1, W2, bconv, Wout, bout):
    n, d_in = x.shape
    d_hid = W0.shape[1]
    d_out = Wout.shape[1]
    n_layers = W1.shape[0]
    nsub = d_hid // LANES
    nvw = NTEC * NSPLIT
    npt2 = -(-((n + nvw - 1) // nvw) // 8) * 8
    np_ = nvw * npt2

    row = edge_index[0].astype(jnp.int32)
    col = edge_index[1].astype(jnp.int32)
    loopidx = jnp.arange(n, dtype=jnp.int32)
    row2 = jnp.concatenate([row, loopidx])
    col2 = jnp.concatenate([col, loopidx])
    wall = jnp.concatenate(
        [edge_attr.astype(jnp.float32), jnp.ones((n,), jnp.float32)])
    perm = jnp.argsort(col2)
    rows_s = row2[perm]
    cols_s = col2[perm]
    ws_s = wall[perm]
    e2 = rows_s.shape[0]
    gran = NTEC * LANES
    e2p = -(-(e2 + 2 * CHUNK) // gran) * gran
    padn = e2p - e2
    rows_p = jnp.concatenate([rows_s, jnp.zeros((padn,), jnp.int32)])
    cols_p = jnp.concatenate([cols_s, jnp.zeros((padn,), jnp.int32)])
    ws_p = jnp.concatenate([ws_s, jnp.zeros((padn,), jnp.float32)])
    ec = e2p // NTEC

    bounds = jnp.arange(nvw + 1, dtype=jnp.int32) * npt2
    starts = jnp.searchsorted(cols_s, bounds).astype(jnp.int32)
    starts_p = jnp.concatenate([starts, jnp.zeros((15,), jnp.int32)])
    xp = jnp.pad(x, ((0, np_ - n), (0, 0)))

    mesh = plsc.VectorSubcoreMesh(core_axis_name="c", subcore_axis_name="s",
                                  num_cores=NC, num_subcores=NS)

    scp = pltpu.CompilerParams(needs_layout_passes=False)
    deg_k = pl.kernel(
        functools.partial(_deg_body, np_, ec),
        out_type=jax.ShapeDtypeStruct((NC, np_), jnp.float32),
        mesh=mesh,
        compiler_params=scp,
        scratch_types=[
            pltpu.VMEM((np_,), jnp.float32),
            pltpu.VMEM((ec,), jnp.int32),
            pltpu.VMEM((ec,), jnp.float32),
            pltpu.VMEM((NS, np_ // NS), jnp.float32),
            pltpu.VMEM_SHARED((NS, np_), jnp.float32),
        ],
    )
    dpart = deg_k(rows_p, ws_p)

    norm_k = pl.kernel(
        functools.partial(_norm_body, np_, ec),
        out_type=jax.ShapeDtypeStruct((e2p,), jnp.float32),
        mesh=mesh,
        compiler_params=scp,
        scratch_types=[
            pltpu.VMEM((NC, np_), jnp.float32),
            pltpu.VMEM((np_,), jnp.float32),
            pltpu.VMEM((ec,), jnp.int32),
            pltpu.VMEM((ec,), jnp.int32),
            pltpu.VMEM((ec,), jnp.float32),
            pltpu.VMEM((ec,), jnp.float32),
        ],
    )
    normv = norm_k(rows_p, cols_p, ws_p, dpart)

    spmm_k = pl.kernel(
        functools.partial(_spmm_body, npt2, nsub),
        out_type=jax.ShapeDtypeStruct((np_, d_hid), jnp.float32),
        mesh=mesh,
        compiler_params=scp,
        scratch_types=[
            pltpu.VMEM((npt2, d_hid), jnp.float32),
            pltpu.VMEM((NBUF, CHUNK, d_hid), jnp.float32),
            pltpu.VMEM((NBUF, CHUNK), jnp.int32),
            pltpu.VMEM((NBUF, CHUNK), jnp.int32),
            pltpu.VMEM((NBUF, CHUNK), jnp.float32),
            pltpu.VMEM((nvw + 16,), jnp.int32),
            pltpu.SemaphoreType.DMA((NBUF,)),
            pltpu.SemaphoreType.DMA((NBUF,)),
        ],
    )

    bm = 512 if np_ % 512 == 0 else npt
    grid = (np_ // bm,)
    h0, zs = pl.pallas_call(
        functools.partial(_mlp_body, n_layers),
        grid=grid,
        in_specs=[
            pl.BlockSpec((bm, d_in), lambda i: (i, 0)),
            pl.BlockSpec((d_in, d_hid), lambda i: (0, 0)),
            pl.BlockSpec((1, d_hid), lambda i: (0, 0)),
            pl.BlockSpec((n_layers, d_hid, d_hid), lambda i: (0, 0, 0)),
            pl.BlockSpec((n_layers, d_hid), lambda i: (0, 0)),
        ],
        out_specs=[
            pl.BlockSpec((bm, d_hid), lambda i: (i, 0)),
            pl.BlockSpec((n_layers, bm, d_hid), lambda i: (0, i, 0)),
        ],
        out_shape=[
            jax.ShapeDtypeStruct((np_, d_hid), jnp.float32),
            jax.ShapeDtypeStruct((n_layers, np_, d_hid), jnp.float32),
        ],
    )(xp, W0, b0.reshape(1, d_hid), W2, bconv)

    layer_k = pl.pallas_call(
        _layer_body,
        grid=grid,
        in_specs=[
            pl.BlockSpec((bm, d_hid), lambda i: (i, 0)),
            pl.BlockSpec((d_hid, d_hid), lambda i: (0, 0)),
            pl.BlockSpec((bm, d_hid), lambda i: (i, 0)),
            pl.BlockSpec((bm, d_hid), lambda i: (i, 0)),
        ],
        out_specs=pl.BlockSpec((bm, d_hid), lambda i: (i, 0)),
        out_shape=jax.ShapeDtypeStruct((np_, d_hid), jnp.float32),
    )

    final_k = pl.pallas_call(
        _final_body,
        grid=grid,
        in_specs=[
            pl.BlockSpec((bm, d_hid), lambda i: (i, 0)),
            pl.BlockSpec((d_hid, d_hid), lambda i: (0, 0)),
            pl.BlockSpec((bm, d_hid), lambda i: (i, 0)),
            pl.BlockSpec((bm, d_hid), lambda i: (i, 0)),
            pl.BlockSpec((d_hid, d_out), lambda i: (0, 0)),
            pl.BlockSpec((1, d_out), lambda i: (0, 0)),
        ],
        out_specs=pl.BlockSpec((bm, d_out), lambda i: (i, 0)),
        out_shape=jax.ShapeDtypeStruct((np_, d_out), jnp.float32),
    )

    prev = h0
    for i in range(n_layers):
        agg = spmm_k(prev, rows_p, cols_p, normv, starts_p)
        if i < n_layers - 1:
            prev = layer_k(agg, W1[i], zs[i], prev)
        else:
            out = final_k(agg, W1[i], zs[i], prev, Wout,
                          bout.reshape(1, d_out))
    return out[:n]
